# baseline (speedup 1.0000x reference)
import functools

import jax
import jax.numpy as jnp
from jax import lax
from jax.experimental import pallas as pl
from jax.experimental.pallas import tpu as pltpu
from jax.experimental.pallas import tpu_sc as plsc

N = 10000
E = 320000
R = 8
B = 4
H = 128
OUT = 16

NC = 2
NS = 16
NW = NC * NS
CH = 80
NBUF = 2
NCHUNK = 128
K0 = 200
K1 = 2 * NCHUNK - K0
PT = NCHUNK * CH
EP = PT * NW
NZSUB = 10
ROWS_PER_SUB = N // NZSUB


def _make_edge_pass(D):
    mesh = plsc.VectorSubcoreMesh(core_axis_name="c", subcore_axis_name="s")

    @functools.partial(
        pl.kernel,
        out_type=jax.ShapeDtypeStruct((NC, N, D), jnp.float32),
        mesh=mesh,
        compiler_params=pltpu.CompilerParams(
            use_tc_tiling_on_sc=(D % 128 == 0), needs_layout_passes=False),
        scratch_types=[
            [pltpu.VMEM((3, CH), jnp.int32) for _ in range(NBUF)],
            [pltpu.VMEM((CH,), jnp.int32) for _ in range(NBUF)],
            [pltpu.VMEM((CH, D), jnp.float32) for _ in range(NBUF)],
            pltpu.VMEM_SHARED((N, D), jnp.float32),
            [pltpu.SemaphoreType.DMA for _ in range(NBUF)],
            [pltpu.SemaphoreType.DMA for _ in range(NBUF)],
            [pltpu.SemaphoreType.DMA for _ in range(NBUF)],
        ],
    )
    def k(idx3_h, tab_h, z_h, out_h,
          ib, db2, rows, acc, msem, gsem, ssem):
        cid = lax.axis_index("c")
        sid = lax.axis_index("s")
        c0 = jnp.where(cid == 0, sid * K0, NS * K0 + sid * K1)
        nch = jnp.where(cid == 0, K0, K1)

        @pl.when(sid < NZSUB)
        def _():
            pltpu.sync_copy(z_h, acc.at[pl.ds(sid * ROWS_PER_SUB, ROWS_PER_SUB)])
        plsc.subcore_barrier()

        def start_meta(ci, b):
            pltpu.async_copy(idx3_h.at[c0 + ci], ib[b], msem[b])

        HC = CH // 2

        def issue_gather(ci, b):
            pltpu.make_async_copy(idx3_h.at[c0 + ci], ib[b], msem[b]).wait()
            pltpu.async_copy(tab_h.at[ib[b].at[0, pl.ds(0, HC)]],
                             rows[b].at[pl.ds(0, HC)], gsem[b])
            pltpu.async_copy(tab_h.at[ib[b].at[0, pl.ds(HC, HC)]],
                             rows[b].at[pl.ds(HC, HC)], gsem[b])

        start_meta(0, 0)
        issue_gather(0, 0)
        start_meta(1, 1)

        _dnums = lax.GatherDimensionNumbers(
            offset_dims=(), collapsed_slice_dims=(0,), start_index_map=(0,))

        def visit(ci, b, drain_other, more):
            o = 1 - b
            pltpu.make_async_copy(tab_h.at[ib[b].at[0, pl.ds(0, HC)]],
                                  rows[b].at[pl.ds(0, HC)], gsem[b]).wait()
            pltpu.make_async_copy(tab_h.at[ib[b].at[0, pl.ds(HC, HC)]],
                                  rows[b].at[pl.ds(HC, HC)], gsem[b]).wait()

            def mul_body(g, _):
                w16 = plsc.bitcast(ib[b][2, pl.ds(g * 16, 16)], jnp.float32)
                for j in range(16):
                    e = g * 16 + j
                    wsp = lax.gather(
                        w16, jnp.full((16, 1), j, jnp.int32), _dnums, (1,),
                        mode=lax.GatherScatterMode.PROMISE_IN_BOUNDS)
                    for c in range(D // 16):
                        sl = pl.ds(c * 16, 16)
                        rows[b][e, sl] = rows[b][e, sl] * wsp
                return 0

            for g in range(CH // 16):
                mul_body(g, 0)
            for g2 in range(CH // 16):
                sl = pl.ds(g2 * 16, 16)
                db2[b][sl] = ib[b][1, sl]
            pltpu.async_copy(rows[b], acc.at[db2[b]], ssem[b], add=True)
            @pl.when(drain_other)
            def _():
                pltpu.make_async_copy(tab_h.at[pl.ds(0, CH)], rows[o],
                                      ssem[o]).wait()

            @pl.when(more)
            def _():
                issue_gather(ci + 1, o)

            @pl.when(ci + 2 < nch)
            def _():
                start_meta(ci + 2, b)

        def pair_body(g, _):
            visit(2 * g, 0, g > 0, 2 * g + 1 < nch)
            visit(2 * g + 1, 1, g >= 0, 2 * g + 2 < nch)
            return 0

        lax.fori_loop(0, nch // NBUF, pair_body, 0)
        pltpu.make_async_copy(tab_h.at[pl.ds(0, CH)], rows[1],
                              ssem[1]).wait()
        plsc.subcore_barrier()

        @pl.when(sid < NZSUB)
        def _():
            sl = pl.ds(sid * ROWS_PER_SUB, ROWS_PER_SUB)
            pltpu.sync_copy(acc.at[sl], out_h.at[cid].at[sl])

    return k


_edge_pass128 = _make_edge_pass(H)
_edge_pass16 = _make_edge_pass(OUT)


_T0_CHUNK = 12800


def _t0_body(wc_ref, w0_ref, out_ref):
    acc = wc_ref[:, 0:1] * w0_ref[0:1, :]
    for b in range(1, B):
        acc = acc + wc_ref[:, b:b + 1] * w0_ref[b:b + 1, :]
    out_ref[...] = acc


def _t0_table(wc0, w0flat):
    grid = (N * H) // _T0_CHUNK
    return pl.pallas_call(
        _t0_body,
        grid=(grid,),
        in_specs=[
            pl.BlockSpec((R, B), lambda i: (0, 0)),
            pl.BlockSpec((B, _T0_CHUNK), lambda i: (0, i)),
        ],
        out_specs=pl.BlockSpec((R, _T0_CHUNK), lambda i: (0, i)),
        out_shape=jax.ShapeDtypeStruct((R, N * H), jnp.float32),
    )(wc0, w0flat)


_MM_ROWS = 400


def _layer_mm_body(p_ref, wm_ref, out_ref):
    x = jnp.maximum(p_ref[0] + p_ref[1], 0.0)
    out_ref[...] = jnp.dot(x, wm_ref[...], preferred_element_type=jnp.float32)


def _layer_mm(p, wm):
    ro = wm.shape[1]
    grid = N // _MM_ROWS
    return pl.pallas_call(
        _layer_mm_body,
        grid=(grid,),
        in_specs=[
            pl.BlockSpec((NC, _MM_ROWS, H), lambda i: (0, i, 0)),
            pl.BlockSpec((H, ro), lambda i: (0, 0)),
        ],
        out_specs=pl.BlockSpec((_MM_ROWS, ro), lambda i: (i, 0)),
        out_shape=jax.ShapeDtypeStruct((N, ro), jnp.float32),
    )(p, wm)


def _softmax_body(p_ref, out_ref):
    x = p_ref[0] + p_ref[1]
    m = jnp.max(x, axis=1, keepdims=True)
    e = jnp.exp(x - m)
    out_ref[...] = e / jnp.sum(e, axis=1, keepdims=True)


def _softmax(p):
    grid = N // _MM_ROWS
    return pl.pallas_call(
        _softmax_body,
        grid=(grid,),
        in_specs=[pl.BlockSpec((NC, _MM_ROWS, OUT), lambda i: (0, i, 0))],
        out_specs=pl.BlockSpec((_MM_ROWS, OUT), lambda i: (i, 0)),
        out_shape=jax.ShapeDtypeStruct((N, OUT), jnp.float32),
    )(p)


def _wmat(wc, W, I, O):
    comb = jnp.einsum('rb,bio->iro', wc, W).reshape(I * R, O)
    return comb.reshape(R, I, O).transpose(1, 0, 2).reshape(I, R * O)


def kernel(edge_index, edge_type, edge_weight, W0, wc0, W1, wc1, W2, wc2):
    src = edge_index[0]
    dst = edge_index[1]
    pad = EP - E
    zi = jnp.zeros((pad,), jnp.int32)
    srcp = jnp.concatenate([src, zi])
    typp = jnp.concatenate([edge_type, zi])
    didx = jnp.concatenate([dst, zi]).reshape(NW * NCHUNK, 1, CH)
    wbits = lax.bitcast_convert_type(
        jnp.concatenate([edge_weight, jnp.zeros((pad,), jnp.float32)]),
        jnp.int32).reshape(NW * NCHUNK, 1, CH)
    k0 = typp * N + srcp
    gidx0 = (k0 & (R - 1)) * N + (k0 >> 3)
    gidx12 = srcp * R + typp
    idx0 = jnp.concatenate(
        [gidx0.reshape(NW * NCHUNK, 1, CH), didx, wbits], axis=1)
    idx12 = jnp.concatenate(
        [gidx12.reshape(NW * NCHUNK, 1, CH), didx, wbits], axis=1)
    zH = jnp.zeros((ROWS_PER_SUB, H), jnp.float32)
    zO = jnp.zeros((ROWS_PER_SUB, OUT), jnp.float32)

    t0 = _t0_table(wc0, W0.reshape(B, N * H)).reshape(R * N, H)
    p0 = _edge_pass128(idx0, t0, zH)
    t1 = _layer_mm(p0, _wmat(wc1, W1, H, H)).reshape(N * R, H)
    p1 = _edge_pass128(idx12, t1, zH)
    t2 = _layer_mm(p1, _wmat(wc2, W2, H, OUT)).reshape(N * R, OUT)
    p2 = _edge_pass16(idx12, t2, zO)
    return _softmax(p2)


# speedup vs baseline: 1.0102x; 1.0102x over previous
import functools

import jax
import jax.numpy as jnp
from jax import lax
from jax.experimental import pallas as pl
from jax.experimental.pallas import tpu as pltpu
from jax.experimental.pallas import tpu_sc as plsc

N = 10000
E = 320000
R = 8
B = 4
H = 128
OUT = 16

NC = 2
NS = 16
NW = NC * NS
CH = 80
NBUF = 2
NCHUNK = 128
K0 = 200
K1 = 2 * NCHUNK - K0
PT = NCHUNK * CH
EP = PT * NW
NZSUB = 10
ROWS_PER_SUB = N // NZSUB


def _make_edge_pass(D):
    mesh = plsc.VectorSubcoreMesh(core_axis_name="c", subcore_axis_name="s")

    @functools.partial(
        pl.kernel,
        out_type=jax.ShapeDtypeStruct((NC, N, D), jnp.float32),
        mesh=mesh,
        compiler_params=pltpu.CompilerParams(
            use_tc_tiling_on_sc=(D % 128 == 0), needs_layout_passes=False),
        scratch_types=[
            [pltpu.VMEM((3, CH), jnp.int32) for _ in range(NBUF)],
            [pltpu.VMEM((CH,), jnp.int32) for _ in range(NBUF)],
            [pltpu.VMEM((CH, D), jnp.float32) for _ in range(NBUF)],
            pltpu.VMEM_SHARED((N, D), jnp.float32),
            [pltpu.SemaphoreType.DMA for _ in range(NBUF)],
            [pltpu.SemaphoreType.DMA for _ in range(NBUF)],
            [pltpu.SemaphoreType.DMA for _ in range(NBUF)],
        ],
    )
    def k(idx3_h, tab_h, z_h, out_h,
          ib, db2, rows, acc, msem, gsem, ssem):
        cid = lax.axis_index("c")
        sid = lax.axis_index("s")
        c0 = jnp.where(cid == 0, sid * K0, NS * K0 + sid * K1)
        nch = jnp.where(cid == 0, K0, K1)

        @pl.when(sid < NZSUB)
        def _():
            pltpu.sync_copy(z_h, acc.at[pl.ds(sid * ROWS_PER_SUB, ROWS_PER_SUB)])
        plsc.subcore_barrier()

        def start_meta(ci, b):
            pltpu.async_copy(idx3_h.at[c0 + ci], ib[b], msem[b])

        HC = CH // 2

        def issue_gather(ci, b):
            pltpu.make_async_copy(idx3_h.at[c0 + ci], ib[b], msem[b]).wait()
            pltpu.async_copy(tab_h.at[ib[b].at[0]], rows[b], gsem[b])

        start_meta(0, 0)
        issue_gather(0, 0)
        start_meta(1, 1)

        _dnums = lax.GatherDimensionNumbers(
            offset_dims=(), collapsed_slice_dims=(0,), start_index_map=(0,))

        def visit(ci, b, drain_other, more):
            o = 1 - b
            pltpu.make_async_copy(tab_h.at[ib[b].at[0]], rows[b],
                                  gsem[b]).wait()

            @pl.when(drain_other)
            def _():
                pltpu.make_async_copy(tab_h.at[pl.ds(0, CH)], rows[o],
                                      ssem[o]).wait()

            @pl.when(more)
            def _():
                issue_gather(ci + 1, o)

            def mul_body(g, _):
                w16 = plsc.bitcast(ib[b][2, pl.ds(g * 16, 16)], jnp.float32)
                for j in range(16):
                    e = g * 16 + j
                    wsp = lax.gather(
                        w16, jnp.full((16, 1), j, jnp.int32), _dnums, (1,),
                        mode=lax.GatherScatterMode.PROMISE_IN_BOUNDS)
                    for c in range(D // 16):
                        sl = pl.ds(c * 16, 16)
                        rows[b][e, sl] = rows[b][e, sl] * wsp
                return 0

            for g in range(CH // 16):
                mul_body(g, 0)
            for g2 in range(CH // 16):
                sl = pl.ds(g2 * 16, 16)
                db2[b][sl] = ib[b][1, sl]
            pltpu.async_copy(rows[b], acc.at[db2[b]], ssem[b], add=True)

            @pl.when(ci + 2 < nch)
            def _():
                start_meta(ci + 2, b)

        def pair_body(g, _):
            visit(2 * g, 0, g > 0, 2 * g + 1 < nch)
            visit(2 * g + 1, 1, g >= 0, 2 * g + 2 < nch)
            return 0

        lax.fori_loop(0, nch // NBUF, pair_body, 0)
        pltpu.make_async_copy(tab_h.at[pl.ds(0, CH)], rows[1],
                              ssem[1]).wait()
        plsc.subcore_barrier()

        @pl.when(sid < NZSUB)
        def _():
            sl = pl.ds(sid * ROWS_PER_SUB, ROWS_PER_SUB)
            pltpu.sync_copy(acc.at[sl], out_h.at[cid].at[sl])

    return k


_edge_pass128 = _make_edge_pass(H)
_edge_pass16 = _make_edge_pass(OUT)


_T0_CHUNK = 12800


def _t0_body(wc_ref, w0_ref, out_ref):
    acc = wc_ref[:, 0:1] * w0_ref[0:1, :]
    for b in range(1, B):
        acc = acc + wc_ref[:, b:b + 1] * w0_ref[b:b + 1, :]
    out_ref[...] = acc


def _t0_table(wc0, w0flat):
    grid = (N * H) // _T0_CHUNK
    return pl.pallas_call(
        _t0_body,
        grid=(grid,),
        in_specs=[
            pl.BlockSpec((R, B), lambda i: (0, 0)),
            pl.BlockSpec((B, _T0_CHUNK), lambda i: (0, i)),
        ],
        out_specs=pl.BlockSpec((R, _T0_CHUNK), lambda i: (0, i)),
        out_shape=jax.ShapeDtypeStruct((R, N * H), jnp.float32),
    )(wc0, w0flat)


_MM_ROWS = 400


def _layer_mm_body(p_ref, wm_ref, out_ref):
    x = jnp.maximum(p_ref[0] + p_ref[1], 0.0)
    out_ref[...] = jnp.dot(x, wm_ref[...], preferred_element_type=jnp.float32)


def _layer_mm(p, wm):
    ro = wm.shape[1]
    grid = N // _MM_ROWS
    return pl.pallas_call(
        _layer_mm_body,
        grid=(grid,),
        in_specs=[
            pl.BlockSpec((NC, _MM_ROWS, H), lambda i: (0, i, 0)),
            pl.BlockSpec((H, ro), lambda i: (0, 0)),
        ],
        out_specs=pl.BlockSpec((_MM_ROWS, ro), lambda i: (i, 0)),
        out_shape=jax.ShapeDtypeStruct((N, ro), jnp.float32),
    )(p, wm)


def _softmax_body(p_ref, out_ref):
    x = p_ref[0] + p_ref[1]
    m = jnp.max(x, axis=1, keepdims=True)
    e = jnp.exp(x - m)
    out_ref[...] = e / jnp.sum(e, axis=1, keepdims=True)


def _softmax(p):
    grid = N // _MM_ROWS
    return pl.pallas_call(
        _softmax_body,
        grid=(grid,),
        in_specs=[pl.BlockSpec((NC, _MM_ROWS, OUT), lambda i: (0, i, 0))],
        out_specs=pl.BlockSpec((_MM_ROWS, OUT), lambda i: (i, 0)),
        out_shape=jax.ShapeDtypeStruct((N, OUT), jnp.float32),
    )(p)


def _wmat(wc, W, I, O):
    comb = jnp.einsum('rb,bio->iro', wc, W).reshape(I * R, O)
    return comb.reshape(R, I, O).transpose(1, 0, 2).reshape(I, R * O)


def kernel(edge_index, edge_type, edge_weight, W0, wc0, W1, wc1, W2, wc2):
    src = edge_index[0]
    dst = edge_index[1]
    pad = EP - E
    zi = jnp.zeros((pad,), jnp.int32)
    srcp = jnp.concatenate([src, zi])
    typp = jnp.concatenate([edge_type, zi])
    didx = jnp.concatenate([dst, zi]).reshape(NW * NCHUNK, 1, CH)
    wbits = lax.bitcast_convert_type(
        jnp.concatenate([edge_weight, jnp.zeros((pad,), jnp.float32)]),
        jnp.int32).reshape(NW * NCHUNK, 1, CH)
    k0 = typp * N + srcp
    gidx0 = (k0 & (R - 1)) * N + (k0 >> 3)
    gidx12 = srcp * R + typp
    idx0 = jnp.concatenate(
        [gidx0.reshape(NW * NCHUNK, 1, CH), didx, wbits], axis=1)
    idx12 = jnp.concatenate(
        [gidx12.reshape(NW * NCHUNK, 1, CH), didx, wbits], axis=1)
    zH = jnp.zeros((ROWS_PER_SUB, H), jnp.float32)
    zO = jnp.zeros((ROWS_PER_SUB, OUT), jnp.float32)

    t0 = _t0_table(wc0, W0.reshape(B, N * H)).reshape(R * N, H)
    p0 = _edge_pass128(idx0, t0, zH)
    t1 = _layer_mm(p0, _wmat(wc1, W1, H, H)).reshape(N * R, H)
    p1 = _edge_pass128(idx12, t1, zH)
    t2 = _layer_mm(p1, _wmat(wc2, W2, H, OUT)).reshape(N * R, OUT)
    p2 = _edge_pass16(idx12, t2, zO)
    return _softmax(p2)


# speedup vs baseline: 1.0107x; 1.0005x over previous
import functools

import jax
import jax.numpy as jnp
from jax import lax
from jax.experimental import pallas as pl
from jax.experimental.pallas import tpu as pltpu
from jax.experimental.pallas import tpu_sc as plsc

N = 10000
E = 320000
R = 8
B = 4
H = 128
OUT = 16

NC = 2
NS = 16
NW = NC * NS
CH = 80
NBUF = 2
NCHUNK = 128
K0 = 200
K1 = 2 * NCHUNK - K0
PT = NCHUNK * CH
EP = PT * NW
NZSUB = 10
ROWS_PER_SUB = N // NZSUB


def _make_edge_pass(D):
    mesh = plsc.VectorSubcoreMesh(core_axis_name="c", subcore_axis_name="s")

    @functools.partial(
        pl.kernel,
        out_type=jax.ShapeDtypeStruct((NC, N, D), jnp.float32),
        mesh=mesh,
        compiler_params=pltpu.CompilerParams(
            use_tc_tiling_on_sc=(D % 128 == 0), needs_layout_passes=False),
        scratch_types=[
            [pltpu.VMEM((3, CH), jnp.int32) for _ in range(NBUF)],
            [pltpu.VMEM((CH,), jnp.int32) for _ in range(NBUF)],
            [pltpu.VMEM((CH, D), jnp.float32) for _ in range(NBUF)],
            pltpu.VMEM_SHARED((N, D), jnp.float32),
            [pltpu.SemaphoreType.DMA for _ in range(NBUF)],
            [pltpu.SemaphoreType.DMA for _ in range(NBUF)],
        ],
    )
    def k(idx3_h, tab_h, z_h, out_h,
          ib, db2, rows, acc, msem, gsem):
        cid = lax.axis_index("c")
        sid = lax.axis_index("s")
        c0 = jnp.where(cid == 0, sid * K0, NS * K0 + sid * K1)
        nch = jnp.where(cid == 0, K0, K1)

        @pl.when(sid < NZSUB)
        def _():
            pltpu.sync_copy(z_h, acc.at[pl.ds(sid * ROWS_PER_SUB, ROWS_PER_SUB)])
        plsc.subcore_barrier()

        def start_meta(ci, b):
            pltpu.async_copy(idx3_h.at[c0 + ci], ib[b], msem[b])

        HC = CH // 2

        def issue_gather(ci, b):
            pltpu.make_async_copy(idx3_h.at[c0 + ci], ib[b], msem[b]).wait()
            pltpu.async_copy(tab_h.at[ib[b].at[0]], rows[b], gsem[b])

        start_meta(0, 0)
        issue_gather(0, 0)
        start_meta(1, 1)

        _dnums = lax.GatherDimensionNumbers(
            offset_dims=(), collapsed_slice_dims=(0,), start_index_map=(0,))

        def visit(ci, b, more):
            o = 1 - b
            pltpu.make_async_copy(tab_h.at[ib[b].at[0]], rows[b],
                                  gsem[b]).wait()

            @pl.when(more)
            def _():
                issue_gather(ci + 1, o)

            def mul_body(g, _):
                w16 = plsc.bitcast(ib[b][2, pl.ds(g * 16, 16)], jnp.float32)
                for j in range(16):
                    e = g * 16 + j
                    wsp = lax.gather(
                        w16, jnp.full((16, 1), j, jnp.int32), _dnums, (1,),
                        mode=lax.GatherScatterMode.PROMISE_IN_BOUNDS)
                    for c in range(D // 16):
                        sl = pl.ds(c * 16, 16)
                        rows[b][e, sl] = rows[b][e, sl] * wsp
                return 0

            for g in range(CH // 16):
                mul_body(g, 0)
            for g2 in range(CH // 16):
                sl = pl.ds(g2 * 16, 16)
                db2[b][sl] = ib[b][1, sl]
            pltpu.sync_copy(rows[b], acc.at[db2[b]], add=True)

            @pl.when(ci + 2 < nch)
            def _():
                start_meta(ci + 2, b)

        def pair_body(g, _):
            visit(2 * g, 0, 2 * g + 1 < nch)
            visit(2 * g + 1, 1, 2 * g + 2 < nch)
            return 0

        lax.fori_loop(0, nch // NBUF, pair_body, 0)
        plsc.subcore_barrier()

        @pl.when(sid < NZSUB)
        def _():
            sl = pl.ds(sid * ROWS_PER_SUB, ROWS_PER_SUB)
            pltpu.sync_copy(acc.at[sl], out_h.at[cid].at[sl])

    return k


_edge_pass128 = _make_edge_pass(H)
_edge_pass16 = _make_edge_pass(OUT)


_T0_CHUNK = 12800


def _t0_body(wc_ref, w0_ref, out_ref):
    acc = wc_ref[:, 0:1] * w0_ref[0:1, :]
    for b in range(1, B):
        acc = acc + wc_ref[:, b:b + 1] * w0_ref[b:b + 1, :]
    out_ref[...] = acc


def _t0_table(wc0, w0flat):
    grid = (N * H) // _T0_CHUNK
    return pl.pallas_call(
        _t0_body,
        grid=(grid,),
        in_specs=[
            pl.BlockSpec((R, B), lambda i: (0, 0)),
            pl.BlockSpec((B, _T0_CHUNK), lambda i: (0, i)),
        ],
        out_specs=pl.BlockSpec((R, _T0_CHUNK), lambda i: (0, i)),
        out_shape=jax.ShapeDtypeStruct((R, N * H), jnp.float32),
    )(wc0, w0flat)


_MM_ROWS = 400


def _layer_mm_body(p_ref, wm_ref, out_ref):
    x = jnp.maximum(p_ref[0] + p_ref[1], 0.0)
    out_ref[...] = jnp.dot(x, wm_ref[...], preferred_element_type=jnp.float32)


def _layer_mm(p, wm):
    ro = wm.shape[1]
    grid = N // _MM_ROWS
    return pl.pallas_call(
        _layer_mm_body,
        grid=(grid,),
        in_specs=[
            pl.BlockSpec((NC, _MM_ROWS, H), lambda i: (0, i, 0)),
            pl.BlockSpec((H, ro), lambda i: (0, 0)),
        ],
        out_specs=pl.BlockSpec((_MM_ROWS, ro), lambda i: (i, 0)),
        out_shape=jax.ShapeDtypeStruct((N, ro), jnp.float32),
    )(p, wm)


def _softmax_body(p_ref, out_ref):
    x = p_ref[0] + p_ref[1]
    m = jnp.max(x, axis=1, keepdims=True)
    e = jnp.exp(x - m)
    out_ref[...] = e / jnp.sum(e, axis=1, keepdims=True)


def _softmax(p):
    grid = N // _MM_ROWS
    return pl.pallas_call(
        _softmax_body,
        grid=(grid,),
        in_specs=[pl.BlockSpec((NC, _MM_ROWS, OUT), lambda i: (0, i, 0))],
        out_specs=pl.BlockSpec((_MM_ROWS, OUT), lambda i: (i, 0)),
        out_shape=jax.ShapeDtypeStruct((N, OUT), jnp.float32),
    )(p)


def _wmat(wc, W, I, O):
    comb = jnp.einsum('rb,bio->iro', wc, W).reshape(I * R, O)
    return comb.reshape(R, I, O).transpose(1, 0, 2).reshape(I, R * O)


def kernel(edge_index, edge_type, edge_weight, W0, wc0, W1, wc1, W2, wc2):
    src = edge_index[0]
    dst = edge_index[1]
    pad = EP - E
    zi = jnp.zeros((pad,), jnp.int32)
    srcp = jnp.concatenate([src, zi])
    typp = jnp.concatenate([edge_type, zi])
    didx = jnp.concatenate([dst, zi]).reshape(NW * NCHUNK, 1, CH)
    wbits = lax.bitcast_convert_type(
        jnp.concatenate([edge_weight, jnp.zeros((pad,), jnp.float32)]),
        jnp.int32).reshape(NW * NCHUNK, 1, CH)
    k0 = typp * N + srcp
    gidx0 = (k0 & (R - 1)) * N + (k0 >> 3)
    gidx12 = srcp * R + typp
    idx0 = jnp.concatenate(
        [gidx0.reshape(NW * NCHUNK, 1, CH), didx, wbits], axis=1)
    idx12 = jnp.concatenate(
        [gidx12.reshape(NW * NCHUNK, 1, CH), didx, wbits], axis=1)
    zH = jnp.zeros((ROWS_PER_SUB, H), jnp.float32)
    zO = jnp.zeros((ROWS_PER_SUB, OUT), jnp.float32)

    t0 = _t0_table(wc0, W0.reshape(B, N * H)).reshape(R * N, H)
    p0 = _edge_pass128(idx0, t0, zH)
    t1 = _layer_mm(p0, _wmat(wc1, W1, H, H)).reshape(N * R, H)
    p1 = _edge_pass128(idx12, t1, zH)
    t2 = _layer_mm(p1, _wmat(wc2, W2, H, OUT)).reshape(N * R, OUT)
    p2 = _edge_pass16(idx12, t2, zO)
    return _softmax(p2)
